# baseline (speedup 1.0000x reference)
import dataclasses
import functools

import jax
import jax.numpy as jnp
from jax import lax
from jax.experimental import pallas as pl
from jax.experimental.pallas import tpu as pltpu
from jax.experimental.pallas import tpu_sc as plsc

N = 10000
E = 320000
C = 128
NC = 2
NS = 16
NW = NC * NS
K = 128
NCH = E // K
PCH = 80
NCHP = PCH * NW + 2 * NW
TR = N
NP = 10240
ZCH = 80
NZCH = N // ZCH
L = 16


def _sc_segment_sum(ein_hbm, x_hbm, zrow_hbm, zcnt_hbm, agg_out, cnt_out,
                    idx0, idx1, rows0, rows1, cnt_loc, agg_sh, sem0, sem1):
    cid = lax.axis_index("c")
    sid = lax.axis_index("s")
    wid = sid * NC + cid

    pltpu.sync_copy(zcnt_hbm, cnt_loc)

    @pl.loop(sid, NZCH, step=NS)
    def _(k):
        pltpu.sync_copy(zrow_hbm, agg_sh.at[pl.ds(k * ZCH, ZCH)])

    plsc.subcore_barrier()

    pltpu.sync_copy(ein_hbm.at[wid], idx0)
    pltpu.make_async_copy(x_hbm.at[idx0.at[0]], rows0, sem0).start()
    pltpu.sync_copy(ein_hbm.at[wid + NW], idx1)
    pltpu.make_async_copy(x_hbm.at[idx1.at[0]], rows1, sem1).start()

    ones16 = jnp.ones((L,), jnp.float32)
    bufs = ((idx0, rows0, sem0), (idx1, rows1, sem1))

    @pl.loop(0, PCH // 2)
    def _(i):
        base = wid + 2 * i * NW
        for b, (idxb, rowsb, semb) in enumerate(bufs):
            pltpu.make_async_copy(x_hbm.at[idxb.at[0]], rowsb, semb).wait()
            pltpu.sync_copy(rowsb, agg_sh.at[idxb.at[1]], add=True)
            for j in range(K // L):
                plsc.addupdate_scatter(cnt_loc, [idxb[1, pl.ds(j * L, L)]],
                                       ones16)
            pltpu.sync_copy(ein_hbm.at[base + (b + 2) * NW], idxb)
            pltpu.make_async_copy(x_hbm.at[idxb.at[0]], rowsb, semb).start()

    pltpu.make_async_copy(x_hbm.at[idx0.at[0]], rows0, sem0).wait()
    pltpu.make_async_copy(x_hbm.at[idx1.at[0]], rows1, sem1).wait()

    pltpu.sync_copy(cnt_loc, cnt_out.at[pl.ds(wid * NP, NP)])

    plsc.subcore_barrier()

    @pl.loop(sid, NZCH, step=NS)
    def _(k):
        r0 = k * ZCH
        pltpu.sync_copy(agg_sh.at[pl.ds(r0, ZCH)],
                        agg_out.at[pl.ds(cid * N + r0, ZCH)])


@functools.cache
def _sc_segment_sum_call():
    mesh = plsc.VectorSubcoreMesh(core_axis_name="c", subcore_axis_name="s",
                                  num_cores=NC, num_subcores=NS)
    cp = pltpu.CompilerParams()
    if "needs_layout_passes" in pltpu.CompilerParams.__dataclass_fields__:
        cp = dataclasses.replace(cp, needs_layout_passes=False)
    return pl.kernel(
        _sc_segment_sum,
        out_type=(
            jax.ShapeDtypeStruct((NC * N, C), jnp.float32),
            jax.ShapeDtypeStruct((NW * NP,), jnp.float32),
        ),
        mesh=mesh,
        compiler_params=cp,
        scratch_types=[
            pltpu.VMEM((2, K), jnp.int32),
            pltpu.VMEM((2, K), jnp.int32),
            pltpu.VMEM((K, C), jnp.float32),
            pltpu.VMEM((K, C), jnp.float32),
            pltpu.VMEM((NP,), jnp.float32),
            pltpu.VMEM_SHARED((N + 8, C), jnp.float32),
            pltpu.SemaphoreType.DMA,
            pltpu.SemaphoreType.DMA,
        ],
    )


_BR = 1000
_NBLK = N // _BR


def _dense_body(x_ref, a0_ref, a1_ref, c_ref, wlt_ref, bl_ref,
                wrt_ref, g_ref, b_ref, o_ref):
    cnt_row = jnp.sum(c_ref[0], axis=0, keepdims=True)
    recip_row = 1.0 / jnp.maximum(cnt_row, 1.0)
    rows_i = lax.broadcasted_iota(jnp.int32, (_BR, _BR), 0)
    cols_i = lax.broadcasted_iota(jnp.int32, (_BR, _BR), 1)
    diag = jnp.where(rows_i == cols_i,
                     jnp.broadcast_to(recip_row, (_BR, _BR)), 0.0)
    agg = a0_ref[...] + a1_ref[...]
    aggm = jnp.dot(diag, agg, preferred_element_type=jnp.float32)
    xb = x_ref[...]
    conv = (jnp.dot(aggm, wlt_ref[...], preferred_element_type=jnp.float32)
            + jnp.dot(xb, wrt_ref[...], preferred_element_type=jnp.float32)
            + bl_ref[...])
    mean = jnp.mean(conv, axis=-1, keepdims=True)
    cen = conv - mean
    var = jnp.mean(cen * cen, axis=-1, keepdims=True)
    normed = cen * lax.rsqrt(var + 1e-5) * g_ref[...] + b_ref[...]
    o_ref[...] = xb + jnp.maximum(normed, 0.0)


def _dense(x, aggp, cntp, W_lT, b_l, W_rT, ln_gamma, ln_beta):
    return pl.pallas_call(
        _dense_body,
        grid=(_NBLK,),
        in_specs=[
            pl.BlockSpec((_BR, C), lambda i: (i, 0)),
            pl.BlockSpec((_BR, C), lambda i: (i, 0)),
            pl.BlockSpec((_BR, C), lambda i: (i + _NBLK, 0)),
            pl.BlockSpec((1, NW, _BR), lambda i: (i, 0, 0)),
            pl.BlockSpec((C, C), lambda i: (0, 0)),
            pl.BlockSpec((1, C), lambda i: (0, 0)),
            pl.BlockSpec((C, C), lambda i: (0, 0)),
            pl.BlockSpec((1, C), lambda i: (0, 0)),
            pl.BlockSpec((1, C), lambda i: (0, 0)),
        ],
        out_specs=pl.BlockSpec((_BR, C), lambda i: (i, 0)),
        out_shape=jax.ShapeDtypeStruct((N, C), jnp.float32),
    )(x, aggp, aggp, cntp, W_lT, b_l.reshape(1, C), W_rT,
      ln_gamma.reshape(1, C), ln_beta.reshape(1, C))


def kernel(x, edge_index, W_l, b_l, W_r, ln_gamma, ln_beta):
    src = edge_index[0].astype(jnp.int32)
    dst = edge_index[1].astype(jnp.int32)
    ein = jnp.stack([src.reshape(NCH, K), dst.reshape(NCH, K)], axis=1)
    pad = jnp.full((NCHP - NCH, 2, K), TR, jnp.int32)
    ein = jnp.concatenate([ein, pad], axis=0)
    xp = jnp.concatenate([x, jnp.zeros((8, C), x.dtype)], axis=0)
    zrow = jnp.zeros((ZCH, C), jnp.float32)
    zcnt = jnp.zeros((NP,), jnp.float32)
    aggp, cntp = _sc_segment_sum_call()(ein, xp, zrow, zcnt)
    cnt3d = cntp.reshape(NW, NP)[:, :N].reshape(NW, _NBLK, _BR).transpose(1, 0, 2)
    return _dense(x, aggp, cnt3d, W_l.T, b_l, W_r.T, ln_gamma, ln_beta)


# speedup vs baseline: 3.4658x; 3.4658x over previous
import dataclasses
import functools

import jax
import jax.numpy as jnp
from jax import lax
from jax.experimental import pallas as pl
from jax.experimental.pallas import tpu as pltpu
from jax.experimental.pallas import tpu_sc as plsc

N = 10000
E = 320000
C = 128
NC = 2
NS = 16
NW = NC * NS
EPW = E // NW
K = 128
NFULL = EPW // K
TAIL = EPW - NFULL * K
EPAD = 2 * K
NP = 10240
ZCH = 80
NZCH = N // ZCH
L = 16


def _sc_segment_sum(src_hbm, dst_hbm, x_hbm, zrow_hbm, zcnt_hbm,
                    agg_out, cnt_out, src0, dst0, src1, dst1,
                    rows0, rows1, src_t, dst_t, rows_t, cnt_loc, agg_sh,
                    sem0, sem1):
    cid = lax.axis_index("c")
    sid = lax.axis_index("s")
    wid = sid * NC + cid

    pltpu.sync_copy(zcnt_hbm, cnt_loc)

    @pl.loop(sid, NZCH, step=NS)
    def _(k):
        pltpu.sync_copy(zrow_hbm, agg_sh.at[pl.ds(k * ZCH, ZCH)])

    plsc.subcore_barrier()

    ebase = wid * EPW
    pltpu.sync_copy(src_hbm.at[pl.ds(ebase, K)], src0)
    pltpu.sync_copy(dst_hbm.at[pl.ds(ebase, K)], dst0)
    pltpu.make_async_copy(x_hbm.at[src0], rows0, sem0).start()
    pltpu.sync_copy(src_hbm.at[pl.ds(ebase + K, K)], src1)
    pltpu.sync_copy(dst_hbm.at[pl.ds(ebase + K, K)], dst1)
    pltpu.make_async_copy(x_hbm.at[src1], rows1, sem1).start()

    ones16 = jnp.ones((L,), jnp.float32)
    bufs = ((src0, dst0, rows0, sem0), (src1, dst1, rows1, sem1))

    @pl.loop(0, NFULL // 2)
    def _(i):
        base = ebase + 2 * i * K
        for b, (srcb, dstb, rowsb, semb) in enumerate(bufs):
            pltpu.make_async_copy(x_hbm.at[srcb], rowsb, semb).wait()
            pltpu.sync_copy(rowsb, agg_sh.at[dstb], add=True)
            for j in range(K // L):
                plsc.addupdate_scatter(cnt_loc, [dstb[pl.ds(j * L, L)]],
                                       ones16)
            nbase = base + (b + 2) * K
            pltpu.sync_copy(src_hbm.at[pl.ds(nbase, K)], srcb)
            pltpu.sync_copy(dst_hbm.at[pl.ds(nbase, K)], dstb)
            pltpu.make_async_copy(x_hbm.at[srcb], rowsb, semb).start()

    pltpu.make_async_copy(x_hbm.at[src0], rows0, sem0).wait()
    pltpu.make_async_copy(x_hbm.at[src1], rows1, sem1).wait()

    tbase = ebase + NFULL * K
    pltpu.sync_copy(src_hbm.at[pl.ds(tbase, TAIL)], src_t)
    pltpu.sync_copy(dst_hbm.at[pl.ds(tbase, TAIL)], dst_t)
    pltpu.async_copy(x_hbm.at[src_t], rows_t, sem0).wait()
    pltpu.sync_copy(rows_t, agg_sh.at[dst_t], add=True)
    for j in range(TAIL // L):
        plsc.addupdate_scatter(cnt_loc, [dst_t[pl.ds(j * L, L)]], ones16)

    pltpu.sync_copy(cnt_loc, cnt_out.at[pl.ds(wid * NP, NP)])

    plsc.subcore_barrier()

    @pl.loop(sid, NZCH, step=NS)
    def _(k):
        r0 = k * ZCH
        pltpu.sync_copy(agg_sh.at[pl.ds(r0, ZCH)],
                        agg_out.at[pl.ds(cid * N + r0, ZCH)])


@functools.cache
def _sc_segment_sum_call():
    mesh = plsc.VectorSubcoreMesh(core_axis_name="c", subcore_axis_name="s",
                                  num_cores=NC, num_subcores=NS)
    cp = pltpu.CompilerParams()
    if "needs_layout_passes" in pltpu.CompilerParams.__dataclass_fields__:
        cp = dataclasses.replace(cp, needs_layout_passes=False)
    return pl.kernel(
        _sc_segment_sum,
        out_type=(
            jax.ShapeDtypeStruct((NC * N, C), jnp.float32),
            jax.ShapeDtypeStruct((NW * NP,), jnp.float32),
        ),
        mesh=mesh,
        compiler_params=cp,
        scratch_types=[
            pltpu.VMEM((K,), jnp.int32),
            pltpu.VMEM((K,), jnp.int32),
            pltpu.VMEM((K,), jnp.int32),
            pltpu.VMEM((K,), jnp.int32),
            pltpu.VMEM((K, C), jnp.float32),
            pltpu.VMEM((K, C), jnp.float32),
            pltpu.VMEM((TAIL,), jnp.int32),
            pltpu.VMEM((TAIL,), jnp.int32),
            pltpu.VMEM((TAIL, C), jnp.float32),
            pltpu.VMEM((NP,), jnp.float32),
            pltpu.VMEM_SHARED((N, C), jnp.float32),
            pltpu.SemaphoreType.DMA,
            pltpu.SemaphoreType.DMA,
        ],
    )


_BR = 1000
_NBLK = N // _BR


def _dense_body(x_ref, a0_ref, a1_ref, c_ref, wlt_ref, bl_ref,
                wrt_ref, g_ref, b_ref, o_ref):
    cnt_row = jnp.sum(c_ref[0], axis=0, keepdims=True)
    recip_row = 1.0 / jnp.maximum(cnt_row, 1.0)
    rows_i = lax.broadcasted_iota(jnp.int32, (_BR, _BR), 0)
    cols_i = lax.broadcasted_iota(jnp.int32, (_BR, _BR), 1)
    diag = jnp.where(rows_i == cols_i,
                     jnp.broadcast_to(recip_row, (_BR, _BR)), 0.0)
    agg = a0_ref[...] + a1_ref[...]
    aggm = jnp.dot(diag, agg, preferred_element_type=jnp.float32)
    xb = x_ref[...]
    conv = (jnp.dot(aggm, wlt_ref[...], preferred_element_type=jnp.float32)
            + jnp.dot(xb, wrt_ref[...], preferred_element_type=jnp.float32)
            + bl_ref[...])
    mean = jnp.mean(conv, axis=-1, keepdims=True)
    cen = conv - mean
    var = jnp.mean(cen * cen, axis=-1, keepdims=True)
    normed = cen * lax.rsqrt(var + 1e-5) * g_ref[...] + b_ref[...]
    o_ref[...] = xb + jnp.maximum(normed, 0.0)


def _dense(x, aggp, cntp, W_lT, b_l, W_rT, ln_gamma, ln_beta):
    return pl.pallas_call(
        _dense_body,
        grid=(_NBLK,),
        in_specs=[
            pl.BlockSpec((_BR, C), lambda i: (i, 0)),
            pl.BlockSpec((_BR, C), lambda i: (i, 0)),
            pl.BlockSpec((_BR, C), lambda i: (i + _NBLK, 0)),
            pl.BlockSpec((1, NW, _BR), lambda i: (i, 0, 0)),
            pl.BlockSpec((C, C), lambda i: (0, 0)),
            pl.BlockSpec((1, C), lambda i: (0, 0)),
            pl.BlockSpec((C, C), lambda i: (0, 0)),
            pl.BlockSpec((1, C), lambda i: (0, 0)),
            pl.BlockSpec((1, C), lambda i: (0, 0)),
        ],
        out_specs=pl.BlockSpec((_BR, C), lambda i: (i, 0)),
        out_shape=jax.ShapeDtypeStruct((N, C), jnp.float32),
    )(x, aggp, aggp, cntp, W_lT, b_l.reshape(1, C), W_rT,
      ln_gamma.reshape(1, C), ln_beta.reshape(1, C))


def kernel(x, edge_index, W_l, b_l, W_r, ln_gamma, ln_beta):
    src = edge_index[0].astype(jnp.int32)
    dst = edge_index[1].astype(jnp.int32)
    zpad = jnp.zeros((EPAD,), jnp.int32)
    srcp = jnp.concatenate([src, zpad])
    dstp = jnp.concatenate([dst, zpad])
    zrow = jnp.zeros((ZCH, C), jnp.float32)
    zcnt = jnp.zeros((NP,), jnp.float32)
    aggp, cntp = _sc_segment_sum_call()(srcp, dstp, x, zrow, zcnt)
    cnt3d = cntp.reshape(NW, NP)[:, :N].reshape(NW, _NBLK, _BR).transpose(1, 0, 2)
    return _dense(x, aggp, cnt3d, W_l.T, b_l, W_r.T, ln_gamma, ln_beta)
